# baseline (speedup 1.0000x reference)
import jax
import jax.numpy as jnp
from jax import lax
from jax.experimental import pallas as pl
from jax.experimental.pallas import tpu as pltpu
from jax.experimental.pallas import tpu_sc as plsc

NC, NS = 2, 16
NW = NC * NS
COLS = 128
CHUNK = 64
SPLIT = COLS // CHUNK
NBUF = 10


def _gather_body(x_hbm, idx_hbm, out_hbm, idx_v, rows_v, gsems, ssems):
    wid = lax.axis_index("s") * NC + lax.axis_index("c")
    K, B = idx_hbm.shape
    col0 = wid * COLS
    pltpu.sync_copy(idx_hbm.at[:, pl.ds(col0, COLS)], idx_v)

    def gather_copy(c, b):
        g, h = lax.div(c, SPLIT), lax.rem(c, SPLIT)
        return pltpu.make_async_copy(
            x_hbm.at[idx_v.at[g, pl.ds(h * CHUNK, CHUNK)]], rows_v.at[b],
            gsems.at[b])

    def store_copy(c, b):
        g, h = lax.div(c, SPLIT), lax.rem(c, SPLIT)
        return pltpu.make_async_copy(
            rows_v.at[b], out_hbm.at[pl.ds(g * B + col0 + h * CHUNK, CHUNK)],
            ssems.at[b])

    for b in range(NBUF):
        gather_copy(b, b).start()

    n_groups = (K * SPLIT) // NBUF

    def group(o, carry):
        for b in range(NBUF):
            g = o * NBUF + b
            gather_copy(g, b).wait()
            store_copy(g, b).start()
            store_copy(g, b).wait()
            gather_copy(g + NBUF, b).start()
        return carry

    lax.fori_loop(0, n_groups - 1, group, 0)

    for b in range(NBUF):
        g = (n_groups - 1) * NBUF + b
        gather_copy(g, b).wait()
        store_copy(g, b).start()
        store_copy(g, b).wait()


def kernel(x, index):
    B, K = index.shape
    D = x.shape[1]
    idx_t = jnp.swapaxes(index, 0, 1).astype(jnp.int32)

    gather = pl.kernel(
        _gather_body,
        out_type=jax.ShapeDtypeStruct((K * B, D), x.dtype),
        mesh=plsc.VectorSubcoreMesh(core_axis_name="c", subcore_axis_name="s",
                                    num_cores=NC, num_subcores=NS),
        scratch_types=[
            pltpu.VMEM((K, COLS), jnp.int32),
            pltpu.VMEM((NBUF, CHUNK, D), jnp.float32),
            pltpu.SemaphoreType.DMA((NBUF,)),
            pltpu.SemaphoreType.DMA((NBUF,)),
        ],
    )
    out_flat = gather(x, idx_t)
    return jnp.swapaxes(out_flat.reshape(K, B, D), 0, 1)


# speedup vs baseline: 1.0110x; 1.0110x over previous
import jax
import jax.numpy as jnp
from jax import lax
from jax.experimental import pallas as pl
from jax.experimental.pallas import tpu as pltpu
from jax.experimental.pallas import tpu_sc as plsc

NC, NS = 2, 16
NW = NC * NS
CHUNK = 128
NBUF = 5


def _gather_body(x_hbm, idx_hbm, out_hbm, idx_v, rows_v, gsems, ssems):
    wid = lax.axis_index("s") * NC + lax.axis_index("c")
    K, B = idx_hbm.shape
    col0 = wid * CHUNK
    pltpu.sync_copy(idx_hbm.at[:, pl.ds(col0, CHUNK)], idx_v)

    def gather_copy(g, b):
        return pltpu.make_async_copy(
            x_hbm.at[idx_v.at[g]], rows_v.at[b], gsems.at[b])

    def store_copy(g, b):
        return pltpu.make_async_copy(
            rows_v.at[b], out_hbm.at[pl.ds(g * B + col0, CHUNK)],
            ssems.at[b])

    for b in range(NBUF):
        gather_copy(b, b).start()

    n_groups = K // NBUF

    def group(o, carry):
        for b in range(NBUF):
            g = o * NBUF + b
            gather_copy(g, b).wait()
            store_copy(g, b).start()
            store_copy(g, b).wait()
            gather_copy(g + NBUF, b).start()
        return carry

    lax.fori_loop(0, n_groups - 1, group, 0)

    for b in range(NBUF):
        g = (n_groups - 1) * NBUF + b
        gather_copy(g, b).wait()
        store_copy(g, b).start()
        store_copy(g, b).wait()


def kernel(x, index):
    B, K = index.shape
    D = x.shape[1]
    idx_t = jnp.swapaxes(index, 0, 1).astype(jnp.int32)

    gather = pl.kernel(
        _gather_body,
        out_type=jax.ShapeDtypeStruct((K * B, D), x.dtype),
        mesh=plsc.VectorSubcoreMesh(core_axis_name="c", subcore_axis_name="s",
                                    num_cores=NC, num_subcores=NS),
        compiler_params=pltpu.CompilerParams(skip_device_barrier=True),
        scratch_types=[
            pltpu.VMEM((K, CHUNK), jnp.int32),
            pltpu.VMEM((NBUF, CHUNK, D), jnp.float32),
            pltpu.SemaphoreType.DMA((NBUF,)),
            pltpu.SemaphoreType.DMA((NBUF,)),
        ],
    )
    out_flat = gather(x, idx_t)
    return jnp.swapaxes(out_flat.reshape(K, B, D), 0, 1)


# speedup vs baseline: 1.0135x; 1.0025x over previous
import jax
import jax.numpy as jnp
from jax import lax
from jax.experimental import pallas as pl
from jax.experimental.pallas import tpu as pltpu
from jax.experimental.pallas import tpu_sc as plsc

NC, NS = 2, 16
NW = NC * NS
CHUNK = 128
NBUF = 7
GD = 4


def _gather_body(x_hbm, idx_hbm, out_hbm, idx_v, rows_v, gsems, ssems):
    wid = lax.axis_index("s") * NC + lax.axis_index("c")
    K, B = idx_hbm.shape
    col0 = wid * CHUNK
    pltpu.sync_copy(idx_hbm.at[:, pl.ds(col0, CHUNK)], idx_v)

    def gather_copy(g, b):
        return pltpu.make_async_copy(
            x_hbm.at[idx_v.at[g]], rows_v.at[b], gsems.at[b])

    def store_copy(g, b):
        return pltpu.make_async_copy(
            rows_v.at[b], out_hbm.at[pl.ds(g * B + col0, CHUNK)],
            ssems.at[b])

    LAG = NBUF - GD
    n_steady = K - 2 * GD
    assert n_steady % NBUF == 0

    for g in range(GD):
        gather_copy(g, g).start()
    for g in range(LAG):
        gather_copy(g, g % NBUF).wait()
        store_copy(g, g % NBUF).start()
        gather_copy(g + GD, (g + GD) % NBUF).start()
    for g in range(LAG, GD):
        gather_copy(g, g % NBUF).wait()
        store_copy(g, g % NBUF).start()
        store_copy(g - LAG, (g - LAG) % NBUF).wait()
        gather_copy(g + GD, (g + GD) % NBUF).start()

    def group(o, carry):
        for j in range(NBUF):
            g = o * NBUF + j + GD
            b = (GD + j) % NBUF
            gather_copy(g, b).wait()
            store_copy(g, b).start()
            store_copy(g - LAG, (b - LAG) % NBUF).wait()
            gather_copy(g + GD, (b + GD) % NBUF).start()
        return carry

    lax.fori_loop(0, n_steady // NBUF, group, 0)

    for g in range(K - GD, K):
        gather_copy(g, g % NBUF).wait()
        store_copy(g, g % NBUF).start()
    for g in range(K - NBUF, K):
        store_copy(g, g % NBUF).wait()


def kernel(x, index):
    B, K = index.shape
    D = x.shape[1]
    idx_t = jnp.swapaxes(index, 0, 1).astype(jnp.int32)

    gather = pl.kernel(
        _gather_body,
        out_type=jax.ShapeDtypeStruct((K * B, D), x.dtype),
        mesh=plsc.VectorSubcoreMesh(core_axis_name="c", subcore_axis_name="s",
                                    num_cores=NC, num_subcores=NS),
        scratch_types=[
            pltpu.VMEM((K, CHUNK), jnp.int32),
            pltpu.VMEM((NBUF, CHUNK, D), jnp.float32),
            pltpu.SemaphoreType.DMA((NBUF,)),
            pltpu.SemaphoreType.DMA((NBUF,)),
        ],
    )
    out_flat = gather(x, idx_t)
    return jnp.swapaxes(out_flat.reshape(K, B, D), 0, 1)
